# baseline (speedup 1.0000x reference)
import functools

import jax
import jax.numpy as jnp
from jax import lax
from jax.experimental import pallas as pl
from jax.experimental.pallas import tpu as pltpu
from jax.experimental.pallas import tpu_sc as plsc


def _sc_gather(x2d, idx, n_rows, d_model):
    info = plsc.get_sparse_core_info()
    nw = info.num_cores * info.num_subcores
    rpw = n_rows // nw
    mesh = plsc.VectorSubcoreMesh(core_axis_name="c", subcore_axis_name="s")

    @functools.partial(
        pl.kernel,
        mesh=mesh,
        out_type=jax.ShapeDtypeStruct((n_rows, d_model), jnp.float32),
        scratch_types=[
            pltpu.VMEM((rpw,), jnp.int32),
            pltpu.VMEM((rpw, d_model), jnp.float32),
            pltpu.SemaphoreType.DMA,
        ],
    )
    def k(x_hbm, idx_hbm, out_hbm, idx_v, rows_v, sem):
        wid = lax.axis_index("s") * info.num_cores + lax.axis_index("c")
        base = wid * rpw
        pltpu.sync_copy(idx_hbm.at[pl.ds(base, rpw)], idx_v)
        pltpu.async_copy(x_hbm.at[idx_v], rows_v, sem).wait()
        pltpu.sync_copy(rows_v, out_hbm.at[pl.ds(base, rpw)])

    return k(x2d, idx)


def _tc_ffn_combine(inp, ew, idx, w1, w2, w3, seq_len):
    n_experts, cap, d_model = inp.shape
    f_dim = w1.shape[-1]

    def body(idx_ref, inp_ref, ew_ref, w1_ref, w2_ref, w3_ref, out_ref, g_acc, h_acc):
        e = pl.program_id(0)
        f = pl.program_id(1)

        @pl.when((e == 0) & (f == 0))
        def _init():
            out_ref[...] = jnp.zeros_like(out_ref)

        xin = inp_ref[0]

        @pl.when(f == 0)
        def _gate():
            g_acc[...] = jnp.dot(xin, w1_ref[0], preferred_element_type=jnp.float32)

        @pl.when(f == 1)
        def _value():
            g = g_acc[...]
            v = jnp.dot(xin, w2_ref[0], preferred_element_type=jnp.float32)
            h_acc[...] = (g * jax.nn.sigmoid(g)) * v

        @pl.when(f == 2)
        def _down():
            o = jnp.dot(h_acc[...], w3_ref[0], preferred_element_type=jnp.float32)
            weighted = o * ew_ref[0, 0][:, None]
            for c in range(cap):
                i = idx_ref[e, c]
                out_ref[pl.ds(i, 1), :] += weighted[c : c + 1, :]

    return pl.pallas_call(
        body,
        grid=(n_experts, 3),
        in_specs=[
            pl.BlockSpec(memory_space=pltpu.SMEM),
            pl.BlockSpec((1, cap, d_model), lambda e, f: (e, 0, 0)),
            pl.BlockSpec((1, 1, cap), lambda e, f: (e, 0, 0)),
            pl.BlockSpec(
                (1, d_model, f_dim),
                lambda e, f: (jnp.minimum(e + (f >= 1), n_experts - 1), 0, 0),
            ),
            pl.BlockSpec(
                (1, d_model, f_dim),
                lambda e, f: (jnp.minimum(e + (f >= 2), n_experts - 1), 0, 0),
            ),
            pl.BlockSpec((1, f_dim, d_model), lambda e, f: (e, 0, 0)),
        ],
        out_specs=pl.BlockSpec((seq_len, d_model), lambda e, f: (0, 0)),
        out_shape=jax.ShapeDtypeStruct((seq_len, d_model), jnp.float32),
        scratch_shapes=[
            pltpu.VMEM((cap, f_dim), jnp.float32),
            pltpu.VMEM((cap, f_dim), jnp.float32),
        ],
    )(idx, inp, ew, w1, w2, w3)


def kernel(x, expert_weights, token_indices, w1, w2, w3):
    batch, seq_len, d_model = x.shape
    _, n_experts, cap = token_indices.shape
    n_rows = n_experts * cap

    x2d = x.reshape(seq_len, d_model)
    idx2d = token_indices.reshape(n_experts, cap).astype(jnp.int32)

    gathered = _sc_gather(x2d, idx2d.reshape(n_rows), n_rows, d_model)
    out = _tc_ffn_combine(
        gathered.reshape(n_experts, cap, d_model),
        expert_weights.reshape(n_experts, 1, cap),
        idx2d,
        w1,
        w2,
        w3,
        seq_len,
    )
    return out.reshape(batch, seq_len, d_model)


# speedup vs baseline: 1.0424x; 1.0424x over previous
import functools

import jax
import jax.numpy as jnp
from jax import lax
from jax.experimental import pallas as pl
from jax.experimental.pallas import tpu as pltpu
from jax.experimental.pallas import tpu_sc as plsc


def _sc_gather(x2d, idx, n_rows, d_model):
    info = plsc.get_sparse_core_info()
    nw = info.num_cores * info.num_subcores
    rpw = n_rows // nw
    mesh = plsc.VectorSubcoreMesh(core_axis_name="c", subcore_axis_name="s")

    half = rpw // 2

    @functools.partial(
        pl.kernel,
        mesh=mesh,
        out_type=jax.ShapeDtypeStruct((n_rows, d_model), jnp.float32),
        scratch_types=[
            pltpu.VMEM((rpw,), jnp.int32),
            pltpu.VMEM((rpw, d_model), jnp.float32),
            pltpu.SemaphoreType.DMA,
            pltpu.SemaphoreType.DMA,
            pltpu.SemaphoreType.DMA,
        ],
    )
    def k(x_hbm, idx_hbm, out_hbm, idx_v, rows_v, sem_a, sem_b, sem_w):
        wid = lax.axis_index("s") * info.num_cores + lax.axis_index("c")
        base = wid * rpw
        pltpu.sync_copy(idx_hbm.at[pl.ds(base, rpw)], idx_v)
        ga = pltpu.async_copy(
            x_hbm.at[idx_v.at[pl.ds(0, half)]], rows_v.at[pl.ds(0, half)], sem_a
        )
        gb = pltpu.async_copy(
            x_hbm.at[idx_v.at[pl.ds(half, half)]],
            rows_v.at[pl.ds(half, half)],
            sem_b,
        )
        ga.wait()
        wa = pltpu.async_copy(
            rows_v.at[pl.ds(0, half)], out_hbm.at[pl.ds(base, half)], sem_w
        )
        gb.wait()
        pltpu.sync_copy(
            rows_v.at[pl.ds(half, half)], out_hbm.at[pl.ds(base + half, half)]
        )
        wa.wait()

    return k(x2d, idx)


_F_SPLIT = 2


def _tc_ffn_combine(inp, ew, idx, w1, w2, w3, seq_len):
    n_experts, cap, d_model = inp.shape
    f_dim = w1.shape[-1]
    f_blk = f_dim // _F_SPLIT

    def body(idx_ref, inp_ref, ew_ref, w1_ref, w2_ref, w3_ref, out_ref, acc):
        e = pl.program_id(0)
        f = pl.program_id(1)

        @pl.when((e == 0) & (f == 0))
        def _init():
            out_ref[...] = jnp.zeros_like(out_ref)

        xin = inp_ref[0]
        g = jnp.dot(xin, w1_ref[0], preferred_element_type=jnp.float32)
        v = jnp.dot(xin, w2_ref[0], preferred_element_type=jnp.float32)
        h = (g * jax.nn.sigmoid(g)) * v
        o = jnp.dot(h, w3_ref[0], preferred_element_type=jnp.float32)

        @pl.when(f == 0)
        def _set():
            acc[...] = o

        @pl.when(f != 0)
        def _acc():
            acc[...] += o

        @pl.when(f == _F_SPLIT - 1)
        def _combine():
            weighted = acc[...] * ew_ref[0, 0][:, None]
            for c in range(cap):
                i = idx_ref[e, c]
                out_ref[pl.ds(i, 1), :] += weighted[c : c + 1, :]

    return pl.pallas_call(
        body,
        grid=(n_experts, _F_SPLIT),
        in_specs=[
            pl.BlockSpec(memory_space=pltpu.SMEM),
            pl.BlockSpec((1, cap, d_model), lambda e, f: (e, 0, 0)),
            pl.BlockSpec((1, 1, cap), lambda e, f: (e, 0, 0)),
            pl.BlockSpec((1, d_model, f_blk), lambda e, f: (e, 0, f)),
            pl.BlockSpec((1, d_model, f_blk), lambda e, f: (e, 0, f)),
            pl.BlockSpec((1, f_blk, d_model), lambda e, f: (e, f, 0)),
        ],
        out_specs=pl.BlockSpec((seq_len, d_model), lambda e, f: (0, 0)),
        out_shape=jax.ShapeDtypeStruct((seq_len, d_model), jnp.float32),
        scratch_shapes=[pltpu.VMEM((cap, d_model), jnp.float32)],
    )(idx, inp, ew, w1, w2, w3)


def kernel(x, expert_weights, token_indices, w1, w2, w3):
    batch, seq_len, d_model = x.shape
    _, n_experts, cap = token_indices.shape
    n_rows = n_experts * cap

    x2d = x.reshape(seq_len, d_model)
    idx2d = token_indices.reshape(n_experts, cap).astype(jnp.int32)

    gathered = _sc_gather(x2d, idx2d.reshape(n_rows), n_rows, d_model)
    out = _tc_ffn_combine(
        gathered.reshape(n_experts, cap, d_model),
        expert_weights.reshape(n_experts, 1, cap),
        idx2d,
        w1,
        w2,
        w3,
        seq_len,
    )
    return out.reshape(batch, seq_len, d_model)
